# baseline (speedup 1.0000x reference)
import functools

import jax
import jax.numpy as jnp
from jax import lax
from jax.experimental import pallas as pl
from jax.experimental.pallas import tpu as pltpu
from jax.experimental.pallas import tpu_sc as plsc

B = 16384
D = 32
OUT_D = 128
FOLD = 4
SMALL_ROWS = (2, 7, 21)
NQ = 4


def _rsqrt_nr(s):
    i = plsc.bitcast(s, jnp.int32)
    i = jnp.int32(0x5F3759DF) - jnp.right_shift(i, 1)
    y = plsc.bitcast(i, jnp.float32)
    for _ in range(3):
        y = y * (1.5 - 0.5 * s * y * y)
    return y


def _build_sc_kernel():
    info = plsc.get_sparse_core_info()
    nc, ns, nl = info.num_cores, info.num_subcores, info.num_lanes
    nw = nc * ns
    bpw = B // nw
    q = bpw // NQ
    mesh = plsc.VectorSubcoreMesh(core_axis_name="c", subcore_axis_name="s")

    scratch = [
        pltpu.VMEM((bpw,), jnp.int32),
        pltpu.VMEM((bpw,), jnp.int32),
        pltpu.VMEM((q, OUT_D), jnp.float32),
        pltpu.VMEM((q, OUT_D), jnp.float32),
        pltpu.VMEM((q, OUT_D), jnp.float32),
        pltpu.SemaphoreType.DMA,
        pltpu.SemaphoreType.DMA,
    ]
    for v in SMALL_ROWS:
        scratch.append(pltpu.VMEM((bpw,), jnp.int32))
        scratch.append(pltpu.VMEM((v, OUT_D), jnp.float32))

    @functools.partial(
        pl.kernel,
        mesh=mesh,
        out_type=jax.ShapeDtypeStruct((B, OUT_D), jnp.float32),
        scratch_types=scratch,
        compiler_params=pltpu.CompilerParams(needs_layout_passes=False),
    )
    def fused(g_idx, a_idx, o_idx, r_idx, w_g, w_a, w_o, w_r4, out,
              r_idx_v, r_row_v, r_bufa, r_bufb, out_v, sema, semb,
              g_idx_v, g_tab_v, a_idx_v, a_tab_v, o_idx_v, o_tab_v):
        wid = lax.axis_index("s") * nc + lax.axis_index("c")
        base = wid * bpw
        lane = lax.iota(jnp.int32, nl)

        pltpu.sync_copy(r_idx.at[pl.ds(base, bpw)], r_idx_v)
        def rowid_body(c, _):
            idx16 = r_idx_v[pl.ds(c * nl, nl)]
            r_row_v[pl.ds(c * nl, nl)] = jnp.right_shift(idx16, 2)
            return 0
        lax.fori_loop(0, bpw // nl, rowid_body, 0)

        bufs = (r_bufa, r_bufb)
        sems = (sema, semb)
        def fire(i):
            return pltpu.async_copy(
                w_r4.at[r_row_v.at[pl.ds(i * q, q)]], bufs[i % 2], sems[i % 2])
        dmas = {0: fire(0), 1: fire(1)}

        small = (
            (g_idx, w_g, g_idx_v, g_tab_v),
            (a_idx, w_a, a_idx_v, a_tab_v),
            (o_idx, w_o, o_idx_v, o_tab_v),
        )
        for idx_hbm, tab_hbm, idx_v, tab_v in small:
            pltpu.sync_copy(idx_hbm.at[pl.ds(base, bpw)], idx_v)
            pltpu.sync_copy(tab_hbm, tab_v)

        cols = [jnp.full((nl,), j, jnp.int32) for j in range(D)]

        def lookup_normalize(idx_v, tab_v, qi, col_off, area):
            def chunk(c, _):
                row16 = c * nl + lane
                idx16 = idx_v[pl.ds(qi * q + c * nl, nl)]
                if area:
                    off16 = jnp.bitwise_and(idx16, 3) * D
                vals = []
                acc = jnp.zeros((nl,), jnp.float32)
                for j in range(D):
                    if area:
                        v = plsc.load_gather(tab_v, [row16, off16 + j])
                    else:
                        v = plsc.load_gather(tab_v, [idx16, cols[j]])
                    vals.append(v)
                    acc = acc + v * v
                inv = _rsqrt_nr(jnp.maximum(acc, 1e-14))
                scale = jnp.minimum(1.0, inv)
                for j in range(D):
                    plsc.store_scatter(out_v, [row16, cols[j] + col_off],
                                       vals[j] * scale)
                return 0
            lax.fori_loop(0, q // nl, chunk, 0)

        for qi in range(NQ):
            for t, (idx_hbm, tab_hbm, idx_v, tab_v) in enumerate(small):
                lookup_normalize(idx_v, tab_v, qi, t * D, area=False)
            dmas[qi].wait()
            lookup_normalize(r_idx_v, bufs[qi % 2], qi, 3 * D, area=True)
            if qi + 2 < NQ:
                dmas[qi + 2] = fire(qi + 2)
            pltpu.sync_copy(out_v, out.at[pl.ds(base + qi * q, q)])

    return fused


_sc_kernel = _build_sc_kernel()


def kernel(gender_idx, age_idx, occupation_idx, area_idx,
           W_gender, W_age, W_occupation, W_area):
    pad = ((0, 0), (0, OUT_D - D))
    return _sc_kernel(
        gender_idx.astype(jnp.int32),
        age_idx.astype(jnp.int32),
        occupation_idx.astype(jnp.int32),
        area_idx.astype(jnp.int32),
        jnp.pad(W_gender, pad),
        jnp.pad(W_age, pad),
        jnp.pad(W_occupation, pad),
        W_area.reshape(W_area.shape[0] // FOLD, FOLD * D),
    )


# speedup vs baseline: 1.0142x; 1.0142x over previous
import functools

import jax
import jax.numpy as jnp
from jax import lax
from jax.experimental import pallas as pl
from jax.experimental.pallas import tpu as pltpu
from jax.experimental.pallas import tpu_sc as plsc

B = 16384
D = 32
OUT_D = 128
FOLD = 4
SMALL_ROWS = (2, 7, 21)
NQ = 4


def _rsqrt_nr(s):
    i = plsc.bitcast(s, jnp.int32)
    i = jnp.int32(0x5F3759DF) - jnp.right_shift(i, 1)
    y = plsc.bitcast(i, jnp.float32)
    for _ in range(3):
        y = y * (1.5 - 0.5 * s * y * y)
    return y


def _build_sc_kernel():
    info = plsc.get_sparse_core_info()
    nc, ns, nl = info.num_cores, info.num_subcores, info.num_lanes
    nw = nc * ns
    bpw = B // nw
    q = bpw // NQ
    mesh = plsc.VectorSubcoreMesh(core_axis_name="c", subcore_axis_name="s")

    scratch = [
        pltpu.VMEM((4 * bpw,), jnp.int32),
        pltpu.VMEM((bpw,), jnp.int32),
        pltpu.VMEM((q, OUT_D), jnp.float32),
        pltpu.VMEM((q, OUT_D), jnp.float32),
        pltpu.VMEM((q, OUT_D), jnp.float32),
        pltpu.VMEM((q, OUT_D), jnp.float32),
        pltpu.SemaphoreType.DMA,
        pltpu.SemaphoreType.DMA,
        pltpu.SemaphoreType.DMA,
    ]
    for v in SMALL_ROWS:
        scratch.append(pltpu.VMEM((v, OUT_D), jnp.float32))

    @functools.partial(
        pl.kernel,
        mesh=mesh,
        out_type=jax.ShapeDtypeStruct((B, OUT_D), jnp.float32),
        scratch_types=scratch,
        compiler_params=pltpu.CompilerParams(needs_layout_passes=False),
    )
    def fused(all_idx, w_g, w_a, w_o, w_r4, out,
              idx_v, r_row_v, r_bufa, r_bufb, out_va, out_vb,
              sema, semb, osem, g_tab_v, a_tab_v, o_tab_v):
        wid = lax.axis_index("s") * nc + lax.axis_index("c")
        base = wid * bpw
        lane = lax.iota(jnp.int32, nl)

        for t in range(4):
            pltpu.sync_copy(all_idx.at[pl.ds(t * B + base, bpw)],
                            idx_v.at[pl.ds(t * bpw, bpw)])
        def rowid_body(c, _):
            idx16 = idx_v[pl.ds(3 * bpw + c * nl, nl)]
            r_row_v[pl.ds(c * nl, nl)] = jnp.right_shift(idx16, 2)
            return 0
        lax.fori_loop(0, bpw // nl, rowid_body, 0)

        bufs = (r_bufa, r_bufb)
        sems = (sema, semb)
        outs = (out_va, out_vb)
        def fire(i):
            return pltpu.async_copy(
                w_r4.at[r_row_v.at[pl.ds(i * q, q)]], bufs[i % 2], sems[i % 2])
        dmas = {0: fire(0), 1: fire(1)}

        for tab_hbm, tab_v in ((w_g, g_tab_v), (w_a, a_tab_v), (w_o, o_tab_v)):
            pltpu.sync_copy(tab_hbm, tab_v)

        cols = [jnp.full((nl,), j, jnp.int32) for j in range(D)]

        def lookup_normalize(idx_off, tab_v, qi, col_off, out_v, area):
            def chunk(c, _):
                row16 = c * nl + lane
                idx16 = idx_v[pl.ds(idx_off + qi * q + c * nl, nl)]
                if area:
                    off16 = jnp.bitwise_and(idx16, 3) * D
                vals = []
                acc = jnp.zeros((nl,), jnp.float32)
                for j in range(D):
                    if area:
                        v = plsc.load_gather(tab_v, [row16, off16 + j])
                    else:
                        v = plsc.load_gather(tab_v, [idx16, cols[j]])
                    vals.append(v)
                    acc = acc + v * v
                inv = _rsqrt_nr(jnp.maximum(acc, 1e-14))
                scale = jnp.minimum(1.0, inv)
                for j in range(D):
                    plsc.store_scatter(out_v, [row16, cols[j] + col_off],
                                       vals[j] * scale)
                return 0
            lax.fori_loop(0, q // nl, chunk, 0)

        owrites = {}
        for qi in range(NQ):
            out_v = outs[qi % 2]
            if qi >= 2:
                owrites[qi - 2].wait()
            for t, tab_v in enumerate((g_tab_v, a_tab_v, o_tab_v)):
                lookup_normalize(t * bpw, tab_v, qi, t * D, out_v, area=False)
            dmas[qi].wait()
            lookup_normalize(3 * bpw, bufs[qi % 2], qi, 3 * D, out_v, area=True)
            if qi + 2 < NQ:
                dmas[qi + 2] = fire(qi + 2)
            owrites[qi] = pltpu.async_copy(
                out_v, out.at[pl.ds(base + qi * q, q)], osem)
        owrites[NQ - 2].wait()
        owrites[NQ - 1].wait()

    return fused


_sc_kernel = _build_sc_kernel()


def kernel(gender_idx, age_idx, occupation_idx, area_idx,
           W_gender, W_age, W_occupation, W_area):
    pad = ((0, 0), (0, OUT_D - D))
    all_idx = jnp.concatenate([
        gender_idx.astype(jnp.int32),
        age_idx.astype(jnp.int32),
        occupation_idx.astype(jnp.int32),
        area_idx.astype(jnp.int32),
    ])
    return _sc_kernel(
        all_idx,
        jnp.pad(W_gender, pad),
        jnp.pad(W_age, pad),
        jnp.pad(W_occupation, pad),
        W_area.reshape(W_area.shape[0] // FOLD, FOLD * D),
    )


# speedup vs baseline: 1.0314x; 1.0170x over previous
import functools

import jax
import jax.numpy as jnp
from jax import lax
from jax.experimental import pallas as pl
from jax.experimental.pallas import tpu as pltpu
from jax.experimental.pallas import tpu_sc as plsc

B = 16384
D = 32
OUT_D = 128
FOLD = 4
SMALL_ROWS = (2, 7, 21)
NQ = 4


def _rsqrt_nr(s):
    i = plsc.bitcast(s, jnp.int32)
    i = jnp.int32(0x5F3759DF) - jnp.right_shift(i, 1)
    y = plsc.bitcast(i, jnp.float32)
    for _ in range(3):
        y = y * (1.5 - 0.5 * s * y * y)
    return y


def _build_sc_kernel():
    info = plsc.get_sparse_core_info()
    nc, ns, nl = info.num_cores, info.num_subcores, info.num_lanes
    nw = nc * ns
    bpw = B // nw
    q = bpw // NQ
    mesh = plsc.VectorSubcoreMesh(core_axis_name="c", subcore_axis_name="s")

    scratch = [
        pltpu.VMEM((4 * bpw,), jnp.int32),
        pltpu.VMEM((bpw,), jnp.int32),
        pltpu.VMEM((q, OUT_D), jnp.float32),
        pltpu.VMEM((q, OUT_D), jnp.float32),
        pltpu.VMEM((q, OUT_D), jnp.float32),
        pltpu.VMEM((q, OUT_D), jnp.float32),
        pltpu.SemaphoreType.DMA,
        pltpu.SemaphoreType.DMA,
        pltpu.SemaphoreType.DMA,
        pltpu.SemaphoreType.DMA,
        pltpu.SemaphoreType.DMA,
    ]
    for v in SMALL_ROWS:
        scratch.append(pltpu.VMEM((v, OUT_D), jnp.float32))

    @functools.partial(
        pl.kernel,
        mesh=mesh,
        out_type=jax.ShapeDtypeStruct((B, OUT_D), jnp.float32),
        scratch_types=scratch,
        compiler_params=pltpu.CompilerParams(needs_layout_passes=False),
    )
    def fused(all_idx, w_g, w_a, w_o, w_r4, out,
              idx_v, r_row_v, r_bufa, r_bufb, out_va, out_vb,
              sema, semb, osema, osemb, ssem, g_tab_v, a_tab_v, o_tab_v):
        wid = lax.axis_index("s") * nc + lax.axis_index("c")
        base = wid * bpw
        lane = lax.iota(jnp.int32, nl)

        bufs = (r_bufa, r_bufb)
        sems = (sema, semb)
        outs = (out_va, out_vb)
        osems = (osema, osemb)

        pltpu.sync_copy(all_idx.at[pl.ds(3 * B + base, bpw)],
                        idx_v.at[pl.ds(3 * bpw, bpw)])

        def rowids(i):
            def body(c, _):
                idx16 = idx_v[pl.ds(3 * bpw + i * q + c * nl, nl)]
                r_row_v[pl.ds(i * q + c * nl, nl)] = jnp.right_shift(idx16, 2)
                return 0
            lax.fori_loop(0, q // nl, body, 0)

        def fire(i):
            return pltpu.async_copy(
                w_r4.at[r_row_v.at[pl.ds(i * q, q)]], bufs[i % 2], sems[i % 2])

        rowids(0)
        dmas = {0: fire(0)}
        rowids(1)
        dmas[1] = fire(1)
        for i in range(2, NQ):
            rowids(i)

        stage = [
            pltpu.async_copy(w_g, g_tab_v, ssem),
            pltpu.async_copy(w_a, a_tab_v, ssem),
            pltpu.async_copy(w_o, o_tab_v, ssem),
        ]
        for t in range(3):
            stage.append(pltpu.async_copy(
                all_idx.at[pl.ds(t * B + base, bpw)],
                idx_v.at[pl.ds(t * bpw, bpw)], ssem))
        for s in stage:
            s.wait()

        cols = [jnp.full((nl,), j, jnp.int32) for j in range(D)]

        def lookup_normalize(idx_off, tab_v, qi, col_off, out_v, area):
            def chunk(c, _):
                row16 = c * nl + lane
                idx16 = idx_v[pl.ds(idx_off + qi * q + c * nl, nl)]
                if area:
                    off16 = jnp.bitwise_and(idx16, 3) * D
                vals = []
                acc = jnp.zeros((nl,), jnp.float32)
                for j in range(D):
                    if area:
                        v = plsc.load_gather(tab_v, [row16, off16 + j])
                    else:
                        v = plsc.load_gather(tab_v, [idx16, cols[j]])
                    vals.append(v)
                    acc = acc + v * v
                inv = _rsqrt_nr(jnp.maximum(acc, 1e-14))
                scale = jnp.minimum(1.0, inv)
                for j in range(D):
                    plsc.store_scatter(out_v, [row16, cols[j] + col_off],
                                       vals[j] * scale)
                return 0
            lax.fori_loop(0, q // nl, chunk, 0)

        owrites = {}
        for qi in range(NQ):
            out_v = outs[qi % 2]
            if qi >= 2:
                owrites[qi - 2].wait()
            for t, tab_v in enumerate((g_tab_v, a_tab_v, o_tab_v)):
                lookup_normalize(t * bpw, tab_v, qi, t * D, out_v, area=False)
            dmas[qi].wait()
            lookup_normalize(3 * bpw, bufs[qi % 2], qi, 3 * D, out_v, area=True)
            if qi + 2 < NQ:
                dmas[qi + 2] = fire(qi + 2)
            owrites[qi] = pltpu.async_copy(
                out_v, out.at[pl.ds(base + qi * q, q)], osems[qi % 2])
        owrites[NQ - 2].wait()
        owrites[NQ - 1].wait()

    return fused


_sc_kernel = _build_sc_kernel()


def kernel(gender_idx, age_idx, occupation_idx, area_idx,
           W_gender, W_age, W_occupation, W_area):
    pad = ((0, 0), (0, OUT_D - D))
    all_idx = jnp.concatenate([
        gender_idx.astype(jnp.int32),
        age_idx.astype(jnp.int32),
        occupation_idx.astype(jnp.int32),
        area_idx.astype(jnp.int32),
    ])
    return _sc_kernel(
        all_idx,
        jnp.pad(W_gender, pad),
        jnp.pad(W_age, pad),
        jnp.pad(W_occupation, pad),
        W_area.reshape(W_area.shape[0] // FOLD, FOLD * D),
    )
